# Initial kernel scaffold; baseline (speedup 1.0000x reference)
import functools

import jax
import jax.numpy as jnp
from jax import lax
from jax.experimental import pallas as pl
from jax.experimental.pallas import tpu as pltpu
from jax.experimental.pallas import tpu_sc as plsc

TOPK = 32
ROW_LEN = 32768
NUM_ROWS = 32 * 32
LANES = 16
NV_ROW = ROW_LEN // LANES
NCHUNK = 32
CHUNK_VREGS = NV_ROW // NCHUNK

NEG_INF = jnp.float32(-jnp.inf)
BIG_I32 = jnp.int32(2**31 - 1)


def _kernel_body(nw, in_hbm, out_hbm, row_v, cand_v, cand_i, out_v):
  rows_per_w = NUM_ROWS // nw
  info = plsc.get_sparse_core_info()
  nc = info.num_cores
  w = lax.axis_index("s") * nc + lax.axis_index("c")
  lane = lax.iota(jnp.int32, LANES)

  def process_row(r_local, _):
    row = w * rows_per_w + r_local
    pltpu.sync_copy(in_hbm.at[pl.ds(row * ROW_LEN, ROW_LEN)], row_v)

    def chunk_body(c, thr):
      def vmax_body(j, acc):
        return jnp.maximum(acc, row_v[pl.ds((c * CHUNK_VREGS + j) * LANES,
                                            LANES)])
      acc = lax.fori_loop(0, CHUNK_VREGS, vmax_body,
                          jnp.full((LANES,), NEG_INF, jnp.float32))
      return jnp.minimum(thr, jnp.max(acc))

    thr = lax.fori_loop(0, NCHUNK, chunk_body, jnp.float32(jnp.inf))

    def scan_body(j, ptr):
      v = row_v[pl.ds(j * LANES, LANES)]
      m = v >= thr
      mi = jnp.where(m, jnp.int32(1), jnp.int32(0))
      cnt = jnp.sum(mi)

      def do_append(p):
        pos = p + plsc.cumsum(mi) - 1
        plsc.store_scatter(cand_v, [pos], v, mask=m)
        plsc.store_scatter(cand_i, [pos], j * LANES + lane, mask=m)
        return p + cnt

      return lax.cond(cnt > 0, do_append, lambda p: p, ptr)

    n_cand = lax.fori_loop(0, NV_ROW, scan_body, jnp.int32(0))

    plsc.store_scatter(cand_v, [n_cand + lane],
                       jnp.full((LANES,), NEG_INF, jnp.float32))
    nv = (n_cand + LANES - 1) // LANES

    def extract(t, carry):
      v0, v1, p0, p1 = carry

      def fmax(j, acc):
        return jnp.maximum(acc, cand_v[pl.ds(j * LANES, LANES)])

      m_val = jnp.max(lax.fori_loop(0, nv, fmax,
                                    jnp.full((LANES,), NEG_INF, jnp.float32)))

      def fpos(j, best):
        v = cand_v[pl.ds(j * LANES, LANES)]
        pos = jnp.where(v == m_val, j * LANES + lane, BIG_I32)
        return jnp.minimum(best, jnp.min(pos))

      bp = lax.fori_loop(0, nv, fpos, BIG_I32)

      plsc.store_scatter(cand_v, [jnp.full((LANES,), bp, jnp.int32)],
                         jnp.full((LANES,), NEG_INF, jnp.float32),
                         mask=lane == 0)

      v0 = jnp.where(lane == t, m_val, v0)
      v1 = jnp.where(lane == t - LANES, m_val, v1)
      p0 = jnp.where(lane == t, bp, p0)
      p1 = jnp.where(lane == t - LANES, bp, p1)
      return v0, v1, p0, p1

    zf = jnp.zeros((LANES,), jnp.float32)
    zi = jnp.zeros((LANES,), jnp.int32)
    v0, v1, p0, p1 = lax.fori_loop(0, TOPK, extract, (zf, zf, zi, zi))

    def rank_body(j, carry):
      r0, r1 = carry
      pj = jnp.minimum(
          jnp.min(jnp.where(lane == j, p0, BIG_I32)),
          jnp.min(jnp.where(lane == j - LANES, p1, BIG_I32)))
      r0 = r0 + jnp.where(pj < p0, jnp.int32(1), jnp.int32(0))
      r1 = r1 + jnp.where(pj < p1, jnp.int32(1), jnp.int32(0))
      return r0, r1

    r0, r1 = lax.fori_loop(0, TOPK, rank_body, (zi, zi))

    base = r_local * TOPK
    plsc.store_scatter(out_v, [base + r0], v0)
    plsc.store_scatter(out_v, [base + r1], v1)
    return _

  lax.fori_loop(0, rows_per_w, process_row, jnp.int32(0))
  pltpu.sync_copy(out_v, out_hbm.at[pl.ds(w * rows_per_w * TOPK,
                                          rows_per_w * TOPK)])


def kernel(inputs):
  info = plsc.get_sparse_core_info()
  nw = info.num_cores * info.num_subcores
  rows_per_w = NUM_ROWS // nw
  mesh = plsc.VectorSubcoreMesh(core_axis_name="c", subcore_axis_name="s")
  k = pl.kernel(
      functools.partial(_kernel_body, nw),
      out_type=jax.ShapeDtypeStruct((NUM_ROWS * TOPK,), jnp.float32),
      mesh=mesh,
      scratch_types=[
          pltpu.VMEM((ROW_LEN,), jnp.float32),
          pltpu.VMEM((ROW_LEN + LANES,), jnp.float32),
          pltpu.VMEM((ROW_LEN + LANES,), jnp.int32),
          pltpu.VMEM((rows_per_w * TOPK,), jnp.float32),
      ],
  )
  out = k(inputs.reshape(-1))
  return out.reshape(32, 32, TOPK)


# speedup vs baseline: 8.9857x; 8.9857x over previous
import functools

import jax
import jax.numpy as jnp
from jax import lax
from jax.experimental import pallas as pl
from jax.experimental.pallas import tpu as pltpu
from jax.experimental.pallas import tpu_sc as plsc

TOPK = 32
ROW_LEN = 32768
NUM_ROWS = 32 * 32
LANES = 16
NV_ROW = ROW_LEN // LANES
NCHUNK = 32
CHUNK_VREGS = NV_ROW // NCHUNK

NEG_INF = float("-inf")
BIG_I32 = 2**31 - 1


def _kernel_body(nw, in_hbm, out_hbm, row_v, cand_v, cand_i, out_v):
  rows_per_w = NUM_ROWS // nw
  info = plsc.get_sparse_core_info()
  nc = info.num_cores
  w = lax.axis_index("s") * nc + lax.axis_index("c")
  lane = lax.iota(jnp.int32, LANES)

  def process_row(r_local, _):
    row = w * rows_per_w + r_local
    pltpu.sync_copy(in_hbm.at[pl.ds(row * ROW_LEN, ROW_LEN)], row_v)

    def chunk_body(c, thr):
      def vmax_body(j, acc):
        return jnp.maximum(acc, row_v[pl.ds((c * CHUNK_VREGS + j) * LANES,
                                            LANES)])
      acc = lax.fori_loop(0, CHUNK_VREGS, vmax_body,
                          jnp.full((LANES,), NEG_INF, jnp.float32))
      return jnp.minimum(thr, jnp.max(acc))

    thr = lax.fori_loop(0, NCHUNK, chunk_body, jnp.float32(jnp.inf))

    def scan_body(j, ptr):
      v = row_v[pl.ds(j * LANES, LANES)]
      m = v >= thr
      mi = jnp.where(m, jnp.int32(1), jnp.int32(0))
      cnt = jnp.sum(mi)

      def do_append(p):
        pos = p + plsc.cumsum(mi) - 1
        plsc.store_scatter(cand_v, [pos], v, mask=m)
        plsc.store_scatter(cand_i, [pos], j * LANES + lane, mask=m)
        return p + cnt

      return lax.cond(cnt > 0, do_append, lambda p: p, ptr)

    n_cand = lax.fori_loop(0, NV_ROW, scan_body, jnp.int32(0))

    plsc.store_scatter(cand_v, [n_cand + lane],
                       jnp.full((LANES,), NEG_INF, jnp.float32))
    nv = (n_cand + LANES - 1) // LANES

    def extract(t, carry):
      v0, v1, p0, p1 = carry

      def fmax(j, acc):
        return jnp.maximum(acc, cand_v[pl.ds(j * LANES, LANES)])

      m_val = jnp.max(lax.fori_loop(0, nv, fmax,
                                    jnp.full((LANES,), NEG_INF, jnp.float32)))

      def fpos(j, best):
        v = cand_v[pl.ds(j * LANES, LANES)]
        pos = jnp.where(v == m_val, j * LANES + lane, BIG_I32)
        return jnp.minimum(best, jnp.min(pos))

      bp = lax.fori_loop(0, nv, fpos, jnp.int32(BIG_I32))

      plsc.store_scatter(cand_v, [jnp.full((LANES,), bp, jnp.int32)],
                         jnp.full((LANES,), NEG_INF, jnp.float32),
                         mask=lane == 0)

      v0 = jnp.where(lane == t, m_val, v0)
      v1 = jnp.where(lane == t - LANES, m_val, v1)
      p0 = jnp.where(lane == t, bp, p0)
      p1 = jnp.where(lane == t - LANES, bp, p1)
      return v0, v1, p0, p1

    zf = jnp.zeros((LANES,), jnp.float32)
    zi = jnp.zeros((LANES,), jnp.int32)
    v0, v1, p0, p1 = lax.fori_loop(0, TOPK, extract, (zf, zf, zi, zi))

    def rank_body(j, carry):
      r0, r1 = carry
      pj = jnp.minimum(
          jnp.min(jnp.where(lane == j, p0, BIG_I32)),
          jnp.min(jnp.where(lane == j - LANES, p1, BIG_I32)))
      r0 = r0 + jnp.where(pj < p0, jnp.int32(1), jnp.int32(0))
      r1 = r1 + jnp.where(pj < p1, jnp.int32(1), jnp.int32(0))
      return r0, r1

    r0, r1 = lax.fori_loop(0, TOPK, rank_body, (zi, zi))

    base = r_local * TOPK
    plsc.store_scatter(out_v, [base + r0], v0)
    plsc.store_scatter(out_v, [base + r1], v1)
    return _

  lax.fori_loop(0, rows_per_w, process_row, jnp.int32(0))
  pltpu.sync_copy(out_v, out_hbm.at[pl.ds(w * rows_per_w * TOPK,
                                          rows_per_w * TOPK)])


def kernel(inputs):
  info = plsc.get_sparse_core_info()
  nw = info.num_cores * info.num_subcores
  rows_per_w = NUM_ROWS // nw
  mesh = plsc.VectorSubcoreMesh(core_axis_name="c", subcore_axis_name="s")
  k = pl.kernel(
      functools.partial(_kernel_body, nw),
      out_type=jax.ShapeDtypeStruct((NUM_ROWS * TOPK,), jnp.float32),
      mesh=mesh,
      compiler_params=pltpu.CompilerParams(needs_layout_passes=False),
      scratch_types=[
          pltpu.VMEM((ROW_LEN,), jnp.float32),
          pltpu.VMEM((ROW_LEN + LANES,), jnp.float32),
          pltpu.VMEM((ROW_LEN + LANES,), jnp.int32),
          pltpu.VMEM((rows_per_w * TOPK,), jnp.float32),
      ],
  )
  out = k(inputs.reshape(-1))
  return out.reshape(32, 32, TOPK)


# speedup vs baseline: 31.1530x; 3.4670x over previous
import functools

import jax
import jax.numpy as jnp
from jax import lax
from jax.experimental import pallas as pl
from jax.experimental.pallas import tpu as pltpu
from jax.experimental.pallas import tpu_sc as plsc

TOPK = 32
ROW_LEN = 32768
NUM_ROWS = 32 * 32
LANES = 16
NV_ROW = ROW_LEN // LANES
GROUP = 8
NGROUP = NV_ROW // GROUP
STATIC_CAND_VREGS = 8

NEG_INF = float("-inf")
BIG_I32 = 2**31 - 1


def _tree_max(vs):
  while len(vs) > 1:
    vs = [jnp.maximum(a, b) for a, b in zip(vs[::2], vs[1::2])]
  return vs[0]


def _kernel_body(nw, in_hbm, out_hbm, row_v, cand_v, out_v):
  rows_per_w = NUM_ROWS // nw
  info = plsc.get_sparse_core_info()
  nc = info.num_cores
  w = lax.axis_index("s") * nc + lax.axis_index("c")
  lane = lax.iota(jnp.int32, LANES)

  def process_row(r_local, _):
    row = w * rows_per_w + r_local
    pltpu.sync_copy(in_hbm.at[pl.ds(row * ROW_LEN, ROW_LEN)], row_v)

    ninf = jnp.full((LANES,), NEG_INF, jnp.float32)
    for k in range(STATIC_CAND_VREGS):
      cand_v[pl.ds(k * LANES, LANES)] = ninf

    def p0(g, carry):
      a0, a1 = carry
      vs = [row_v[pl.ds((g * GROUP + k) * LANES, LANES)] for k in range(GROUP)]
      a0 = jnp.maximum(a0, _tree_max(vs[0::2]))
      a1 = jnp.maximum(a1, _tree_max(vs[1::2]))
      return a0, a1

    a0, a1 = lax.fori_loop(0, NGROUP, p0, (ninf, ninf))
    thr = jnp.min(jnp.minimum(a0, a1))

    def p1(g, ptr_vec):
      base = g * GROUP * LANES
      vs = [row_v[pl.ds(base + k * LANES, LANES)] for k in range(GROUP)]
      dirty = jnp.any(_tree_max(vs) >= thr)

      def append(p):
        for v in vs:
          m = v >= thr
          mi = jnp.where(m, jnp.int32(1), jnp.int32(0))
          pos = p + plsc.cumsum(mi) - 1
          plsc.store_scatter(cand_v, [pos], v, mask=m)
          p = p + plsc.all_reduce_population_count(m)
        return p

      return lax.cond(dirty, append, lambda p: p, ptr_vec)

    ptr_vec = lax.fori_loop(0, NGROUP, p1, jnp.zeros((LANES,), jnp.int32))
    n_cand = jnp.max(ptr_vec)

    plsc.store_scatter(cand_v, [n_cand + lane], ninf)
    nv = lax.shift_right_logical(n_cand + LANES - 1, 4)

    def make_round(load_max, load_pos):
      def rnd(t, carry):
        v0, v1, p0_, p1_ = carry
        m_val = jnp.max(load_max())
        bp = jnp.min(load_pos(m_val))
        plsc.store_scatter(cand_v, [jnp.full((LANES,), bp, jnp.int32)],
                           ninf, mask=lane == 0)
        v0 = jnp.where(lane == t, m_val, v0)
        v1 = jnp.where(lane == t - LANES, m_val, v1)
        p0_ = jnp.where(lane == t, bp, p0_)
        p1_ = jnp.where(lane == t - LANES, bp, p1_)
        return v0, v1, p0_, p1_
      return rnd

    zf = jnp.zeros((LANES,), jnp.float32)
    zi = jnp.zeros((LANES,), jnp.int32)
    init = (zf, zf, zi, zi)

    def extract_static(_):
      def load_max():
        return _tree_max([cand_v[pl.ds(k * LANES, LANES)]
                          for k in range(STATIC_CAND_VREGS)])

      def load_pos(m_val):
        ps = []
        for k in range(STATIC_CAND_VREGS):
          v = cand_v[pl.ds(k * LANES, LANES)]
          ps.append(jnp.where(v == m_val, k * LANES + lane,
                              jnp.int32(BIG_I32)))
        while len(ps) > 1:
          ps = [jnp.minimum(a, b) for a, b in zip(ps[::2], ps[1::2])]
        return ps[0]

      return lax.fori_loop(0, TOPK, make_round(load_max, load_pos), init)

    def extract_dynamic(_):
      def load_max():
        def fmax(j, acc):
          return jnp.maximum(acc, cand_v[pl.ds(j * LANES, LANES)])
        return lax.fori_loop(0, nv, fmax, ninf)

      def load_pos(m_val):
        def fpos(j, best):
          v = cand_v[pl.ds(j * LANES, LANES)]
          return jnp.minimum(
              best, jnp.where(v == m_val, j * LANES + lane,
                              jnp.int32(BIG_I32)))
        return lax.fori_loop(0, nv, fpos,
                             jnp.full((LANES,), BIG_I32, jnp.int32))

      return lax.fori_loop(0, TOPK, make_round(load_max, load_pos), init)

    v0, v1, p0_, p1_ = lax.cond(n_cand <= STATIC_CAND_VREGS * LANES,
                                extract_static, extract_dynamic, 0)

    def rank_body(j, carry):
      r0, r1 = carry
      pj = jnp.minimum(
          jnp.min(jnp.where(lane == j, p0_, jnp.int32(BIG_I32))),
          jnp.min(jnp.where(lane == j - LANES, p1_, jnp.int32(BIG_I32))))
      r0 = r0 + jnp.where(pj < p0_, jnp.int32(1), jnp.int32(0))
      r1 = r1 + jnp.where(pj < p1_, jnp.int32(1), jnp.int32(0))
      return r0, r1

    r0, r1 = lax.fori_loop(0, TOPK, rank_body, (zi, zi))

    base = r_local * TOPK
    plsc.store_scatter(out_v, [base + r0], v0)
    plsc.store_scatter(out_v, [base + r1], v1)
    return _

  lax.fori_loop(0, rows_per_w, process_row, jnp.int32(0))
  pltpu.sync_copy(out_v, out_hbm.at[pl.ds(w * rows_per_w * TOPK,
                                          rows_per_w * TOPK)])


def kernel(inputs):
  info = plsc.get_sparse_core_info()
  nw = info.num_cores * info.num_subcores
  rows_per_w = NUM_ROWS // nw
  mesh = plsc.VectorSubcoreMesh(core_axis_name="c", subcore_axis_name="s")
  k = pl.kernel(
      functools.partial(_kernel_body, nw),
      out_type=jax.ShapeDtypeStruct((NUM_ROWS * TOPK,), jnp.float32),
      mesh=mesh,
      compiler_params=pltpu.CompilerParams(needs_layout_passes=False),
      scratch_types=[
          pltpu.VMEM((ROW_LEN,), jnp.float32),
          pltpu.VMEM((ROW_LEN + LANES,), jnp.float32),
          pltpu.VMEM((rows_per_w * TOPK,), jnp.float32),
      ],
  )
  out = k(inputs.reshape(-1))
  return out.reshape(32, 32, TOPK)


# speedup vs baseline: 41.0095x; 1.3164x over previous
import functools

import jax
import jax.numpy as jnp
from jax import lax
from jax.experimental import pallas as pl
from jax.experimental.pallas import tpu as pltpu
from jax.experimental.pallas import tpu_sc as plsc

TOPK = 32
ROW_LEN = 32768
NUM_ROWS = 32 * 32
LANES = 16
NV_ROW = ROW_LEN // LANES
GROUP = 8
NGROUP = NV_ROW // GROUP
STATIC_CAND_VREGS = 8

NEG_INF = float("-inf")
BIG_I32 = 2**31 - 1


def _tree_max(vs):
  while len(vs) > 1:
    vs = [jnp.maximum(a, b) for a, b in zip(vs[::2], vs[1::2])]
  return vs[0]


def _kernel_body(nw, in_hbm, out_hbm, row_a, row_b, cand_v, out_v,
                 sem_a, sem_b):
  rows_per_w = NUM_ROWS // nw
  info = plsc.get_sparse_core_info()
  nc = info.num_cores
  w = lax.axis_index("s") * nc + lax.axis_index("c")
  lane = lax.iota(jnp.int32, LANES)
  row0 = w * rows_per_w

  def process_row(r_local, row_v):
    ninf = jnp.full((LANES,), NEG_INF, jnp.float32)
    for k in range(STATIC_CAND_VREGS):
      cand_v[pl.ds(k * LANES, LANES)] = ninf

    def p0(g, carry):
      a0, a1 = carry
      vs = [row_v[pl.ds((g * GROUP + k) * LANES, LANES)] for k in range(GROUP)]
      a0 = jnp.maximum(a0, _tree_max(vs[0::2]))
      a1 = jnp.maximum(a1, _tree_max(vs[1::2]))
      return a0, a1

    a0, a1 = lax.fori_loop(0, NGROUP, p0, (ninf, ninf))
    thr = jnp.min(jnp.minimum(a0, a1))

    def p1(g, ptr_vec):
      base = g * GROUP * LANES
      vs = [row_v[pl.ds(base + k * LANES, LANES)] for k in range(GROUP)]
      dirty = jnp.any(_tree_max(vs) >= thr)

      def append(p):
        for v in vs:
          m = v >= thr
          mi = jnp.where(m, jnp.int32(1), jnp.int32(0))
          pos = p + plsc.cumsum(mi) - 1
          plsc.store_scatter(cand_v, [pos], v, mask=m)
          p = p + plsc.all_reduce_population_count(m)
        return p

      return lax.cond(dirty, append, lambda p: p, ptr_vec)

    ptr_vec = lax.fori_loop(0, NGROUP, p1, jnp.zeros((LANES,), jnp.int32))
    n_cand = jnp.max(ptr_vec)

    plsc.store_scatter(cand_v, [n_cand + lane], ninf)
    nv = lax.shift_right_logical(n_cand + LANES - 1, 4)

    def make_round(load_max, load_pos):
      def rnd(t, carry):
        v0, v1, p0_, p1_ = carry
        m_val = jnp.max(load_max())
        bp = jnp.min(load_pos(m_val))
        plsc.store_scatter(cand_v, [jnp.full((LANES,), bp, jnp.int32)],
                           ninf, mask=lane == 0)
        v0 = jnp.where(lane == t, m_val, v0)
        v1 = jnp.where(lane == t - LANES, m_val, v1)
        p0_ = jnp.where(lane == t, bp, p0_)
        p1_ = jnp.where(lane == t - LANES, bp, p1_)
        return v0, v1, p0_, p1_
      return rnd

    zf = jnp.zeros((LANES,), jnp.float32)
    zi = jnp.zeros((LANES,), jnp.int32)
    init = (zf, zf, zi, zi)

    def extract_static(_):
      def load_max():
        return _tree_max([cand_v[pl.ds(k * LANES, LANES)]
                          for k in range(STATIC_CAND_VREGS)])

      def load_pos(m_val):
        ps = []
        for k in range(STATIC_CAND_VREGS):
          v = cand_v[pl.ds(k * LANES, LANES)]
          ps.append(jnp.where(v == m_val, k * LANES + lane,
                              jnp.int32(BIG_I32)))
        while len(ps) > 1:
          ps = [jnp.minimum(a, b) for a, b in zip(ps[::2], ps[1::2])]
        return ps[0]

      return lax.fori_loop(0, TOPK, make_round(load_max, load_pos), init)

    def extract_dynamic(_):
      def load_max():
        def fmax(j, acc):
          return jnp.maximum(acc, cand_v[pl.ds(j * LANES, LANES)])
        return lax.fori_loop(0, nv, fmax, ninf)

      def load_pos(m_val):
        def fpos(j, best):
          v = cand_v[pl.ds(j * LANES, LANES)]
          return jnp.minimum(
              best, jnp.where(v == m_val, j * LANES + lane,
                              jnp.int32(BIG_I32)))
        return lax.fori_loop(0, nv, fpos,
                             jnp.full((LANES,), BIG_I32, jnp.int32))

      return lax.fori_loop(0, TOPK, make_round(load_max, load_pos), init)

    v0, v1, p0_, p1_ = lax.cond(n_cand <= STATIC_CAND_VREGS * LANES,
                                extract_static, extract_dynamic, 0)

    def rank_body(j, carry):
      r0, r1 = carry
      pj = jnp.minimum(
          jnp.min(jnp.where(lane == j, p0_, jnp.int32(BIG_I32))),
          jnp.min(jnp.where(lane == j - LANES, p1_, jnp.int32(BIG_I32))))
      r0 = r0 + jnp.where(pj < p0_, jnp.int32(1), jnp.int32(0))
      r1 = r1 + jnp.where(pj < p1_, jnp.int32(1), jnp.int32(0))
      return r0, r1

    r0, r1 = lax.fori_loop(0, TOPK, rank_body, (zi, zi))

    base = r_local * TOPK
    plsc.store_scatter(out_v, [base + r0], v0)
    plsc.store_scatter(out_v, [base + r1], v1)

  pltpu.async_copy(in_hbm.at[row0], row_a, sem_a)

  def pair_body(i, _):
    r_even = 2 * i
    pltpu.make_async_copy(in_hbm.at[row0], row_a, sem_a).wait()
    pltpu.async_copy(in_hbm.at[row0 + r_even + 1], row_b, sem_b)
    process_row(r_even, row_a)
    pltpu.make_async_copy(in_hbm.at[row0], row_b, sem_b).wait()

    @pl.when(r_even + 2 < rows_per_w)
    def _start_next():
      pltpu.async_copy(in_hbm.at[row0 + r_even + 2], row_a, sem_a)

    process_row(r_even + 1, row_b)
    return _

  lax.fori_loop(0, rows_per_w // 2, pair_body, jnp.int32(0))
  pltpu.sync_copy(out_v, out_hbm.at[pl.ds(w * rows_per_w * TOPK,
                                          rows_per_w * TOPK)])


def kernel(inputs):
  info = plsc.get_sparse_core_info()
  nw = info.num_cores * info.num_subcores
  rows_per_w = NUM_ROWS // nw
  mesh = plsc.VectorSubcoreMesh(core_axis_name="c", subcore_axis_name="s")
  k = pl.kernel(
      functools.partial(_kernel_body, nw),
      out_type=jax.ShapeDtypeStruct((NUM_ROWS * TOPK,), jnp.float32),
      mesh=mesh,
      compiler_params=pltpu.CompilerParams(needs_layout_passes=False),
      scratch_types=[
          pltpu.VMEM((ROW_LEN,), jnp.float32),
          pltpu.VMEM((ROW_LEN,), jnp.float32),
          pltpu.VMEM((ROW_LEN + LANES,), jnp.float32),
          pltpu.VMEM((rows_per_w * TOPK,), jnp.float32),
          pltpu.SemaphoreType.DMA,
          pltpu.SemaphoreType.DMA,
      ],
  )
  out = k(inputs.reshape(NUM_ROWS, ROW_LEN))
  return out.reshape(32, 32, TOPK)
